# baseline (speedup 1.0000x reference)
import functools

import jax
import jax.numpy as jnp
from jax import lax
from jax.experimental import pallas as pl
from jax.experimental.pallas import tpu as pltpu
from jax.experimental.pallas import tpu_sc as plsc

N = 10000
E = 160000
D_IN = 128
HID = 64
OUT = 16
NP = 10240
EP = 163840
NW = 32
EPW = EP // NW
CH = 128
NCH = EPW // CH
ET = EP // 16
BR = 160
NBLK = NP // BR


def _sc_mesh():
    return plsc.VectorSubcoreMesh(core_axis_name="c", subcore_axis_name="s")


def _abuild_body(srcp_hbm, dstp_hbm, zhbm, ones_hbm, a_hbm, srcv, dstv, idxb, valb, spa):
    c = lax.axis_index("c")
    s = lax.axis_index("s")
    lane = lax.iota(jnp.int32, 16)
    ebase = s * ET
    pltpu.sync_copy(srcp_hbm.at[pl.ds(ebase, ET)], srcv)
    pltpu.sync_copy(dstp_hbm.at[pl.ds(ebase, ET)], dstv)
    pltpu.sync_copy(ones_hbm, valb)
    trash = jnp.int32(BR * NP)

    def block_body(ib, carry):
        blk = ib * 2 + c
        lo = blk * BR
        r0 = s * 10
        pltpu.sync_copy(zhbm, spa.at[pl.ds(r0 * NP, 10 * NP)])

        @pl.when(s == 0)
        def _():
            pltpu.sync_copy(zhbm.at[pl.ds(0, NP)], spa.at[pl.ds(BR * NP, NP)])

        plsc.subcore_barrier()

        def chunk_scan(q, carry2):
            for g in range(CH // 16):
                sl = pl.ds(q * CH + g * 16, 16)
                sv = srcv[sl]
                dv = dstv[sl]
                rel = sv - lo
                inb = ((rel >= 0) & (rel < BR)
                       & ((ebase + q * CH + g * 16 + lane) < E))
                idxb[pl.ds(g * 16, 16)] = jnp.where(inb, rel * NP + dv, trash)
            pltpu.sync_copy(valb, spa.at[idxb], add=True)
            return carry2

        lax.fori_loop(0, ET // CH, chunk_scan, 0)
        plsc.subcore_barrier()
        pltpu.sync_copy(spa.at[pl.ds(r0 * NP, 10 * NP)],
                        a_hbm.at[pl.ds((lo + r0) * NP, 10 * NP)])
        plsc.subcore_barrier()
        return carry

    lax.fori_loop(0, NBLK // 2, block_body, 0)


def _abuild(srcp, dstp):
    f = pl.kernel(
        _abuild_body,
        mesh=_sc_mesh(),
        out_type=[jax.ShapeDtypeStruct((NP * NP,), jnp.float32)],
        scratch_types=[
            pltpu.VMEM((ET,), jnp.int32),
            pltpu.VMEM((ET,), jnp.int32),
            pltpu.VMEM((CH,), jnp.int32),
            pltpu.VMEM((CH,), jnp.float32),
            pltpu.VMEM_SHARED(((BR + 1) * NP,), jnp.float32),
        ],
    )
    zhbm = jnp.zeros((10 * NP,), jnp.float32)
    ones_hbm = jnp.ones((CH,), jnp.float32)
    return f(srcp, dstp, zhbm, ones_hbm)[0]


def _cvt_body(a32, ab):
    ab[...] = a32[...].astype(jnp.bfloat16)


def _cvt_bf16(a32):
    bm, bn = 512, 2048
    return pl.pallas_call(
        _cvt_body,
        grid=(NP // bm, NP // bn),
        in_specs=[pl.BlockSpec((bm, bn), lambda i, j: (i, j))],
        out_specs=pl.BlockSpec((bm, bn), lambda i, j: (i, j)),
        out_shape=jax.ShapeDtypeStruct((NP, NP), jnp.bfloat16),
    )(a32)


def _adj_body(aij, al, ar, a2o, p1o, p2o, dgo, acc, *, bm, bn):
    i, j, k = pl.program_id(0), pl.program_id(1), pl.program_id(2)
    nj, nk = pl.num_programs(1), pl.num_programs(2)

    @pl.when(k == 0)
    def _():
        acc[...] = jnp.zeros_like(acc)

    acc[...] += jnp.dot(al[...], ar[...], preferred_element_type=jnp.float32)

    @pl.when((j == 0) & (k == 0))
    def _():
        p1o[...] = jnp.zeros_like(p1o)
        p2o[...] = jnp.zeros_like(p2o)
        dgo[...] = jnp.zeros_like(dgo)

    @pl.when(k == nk - 1)
    def _():
        a = aij[...].astype(jnp.float32)
        rows = jax.lax.broadcasted_iota(jnp.int32, (bm, bn), 0) + i * bm
        cols = jax.lax.broadcasted_iota(jnp.int32, (bm, bn), 1) + j * bn
        dmask = (rows == cols).astype(jnp.float32)
        a1 = ((a - dmask) > 0.5).astype(jnp.float32)
        a2 = (acc[...] - a - dmask) > 0.5
        a2o[...] = a2.astype(jnp.int8)
        p1o[...] += jnp.broadcast_to(jnp.sum(a1, axis=1, keepdims=True), p1o.shape)
        p2o[...] += jnp.broadcast_to(
            jnp.sum(a2.astype(jnp.float32), axis=1, keepdims=True), p2o.shape)

        @pl.when(i == j)
        def _():
            dgo[...] += jnp.broadcast_to(
                jnp.sum(a * dmask, axis=1, keepdims=True), dgo.shape)

        @pl.when(j == nj - 1)
        def _():
            d1 = p1o[...]
            p1o[...] = jnp.where(d1 > 0.5, jax.lax.rsqrt(jnp.maximum(d1, 1e-30)), 0.0)
            d2 = p2o[...]
            p2o[...] = jnp.where(d2 > 0.5, jax.lax.rsqrt(jnp.maximum(d2, 1e-30)), 0.0)
            dgo[...] = (dgo[...] - 1.0 > 0.5).astype(jnp.float32)


def _adj_structure(a_bf16, *, np_, bm=1024, bn=1024, bk=512, interpret=False):
    nbi, nbj, nbk = np_ // bm, np_ // bn, np_ // bk
    return pl.pallas_call(
        functools.partial(_adj_body, bm=bm, bn=bn),
        grid=(nbi, nbj, nbk),
        in_specs=[
            pl.BlockSpec((bm, bn), lambda i, j, k: (i, j)),
            pl.BlockSpec((bm, bk), lambda i, j, k: (i, k)),
            pl.BlockSpec((bk, bn), lambda i, j, k: (k, j)),
        ],
        out_specs=[
            pl.BlockSpec((bm, bn), lambda i, j, k: (i, j)),
            pl.BlockSpec((bm, 128), lambda i, j, k: (i, 0)),
            pl.BlockSpec((bm, 128), lambda i, j, k: (i, 0)),
            pl.BlockSpec((bm, 128), lambda i, j, k: (i, 0)),
        ],
        out_shape=[
            jax.ShapeDtypeStruct((np_, np_), jnp.int8),
            jax.ShapeDtypeStruct((np_, 128), jnp.float32),
            jax.ShapeDtypeStruct((np_, 128), jnp.float32),
            jax.ShapeDtypeStruct((np_, 128), jnp.float32),
        ],
        scratch_shapes=[pltpu.VMEM((bm, bn), jnp.float32)],
        compiler_params=pltpu.CompilerParams(
            dimension_semantics=("parallel", "parallel", "arbitrary")),
        interpret=interpret,
    )(a_bf16, a_bf16, a_bf16)


def _mlp_body(xb, w1t, b1, w2t, b2, w3t, b3, pm, we, lg_o, y_o, r0_o):
    xv = xb[...]
    h1 = jnp.maximum(
        jnp.dot(xv, w1t[...], preferred_element_type=jnp.float32) + b1[...], 0.0)
    h2 = jnp.maximum(
        jnp.dot(h1, w2t[...], preferred_element_type=jnp.float32) + b2[...], 0.0)
    lg = jnp.dot(h2, w3t[...], preferred_element_type=jnp.float32) + b3[...]
    lg_o[...] = lg
    y_o[...] = jnp.dot(lg, pm[...], preferred_element_type=jnp.float32)
    r0 = jnp.maximum(
        jnp.dot(xv, we[...], preferred_element_type=jnp.float32), 0.0)
    r0_o[...] = jnp.concatenate(
        [r0, jnp.zeros((r0.shape[0], 128 - HID), jnp.float32)], axis=1)


def _mlp(x, w1t, b1, w2t, b2, w3t, b3, pm, we):
    bm = 1000
    g = N // bm
    return pl.pallas_call(
        _mlp_body,
        grid=(g,),
        in_specs=[
            pl.BlockSpec((bm, D_IN), lambda i: (i, 0)),
            pl.BlockSpec((D_IN, 512), lambda i: (0, 0)),
            pl.BlockSpec((1, 512), lambda i: (0, 0)),
            pl.BlockSpec((512, HID), lambda i: (0, 0)),
            pl.BlockSpec((1, HID), lambda i: (0, 0)),
            pl.BlockSpec((HID, OUT), lambda i: (0, 0)),
            pl.BlockSpec((1, OUT), lambda i: (0, 0)),
            pl.BlockSpec((OUT, OUT), lambda i: (0, 0)),
            pl.BlockSpec((D_IN, HID), lambda i: (0, 0)),
        ],
        out_specs=[
            pl.BlockSpec((bm, OUT), lambda i: (i, 0)),
            pl.BlockSpec((bm, OUT), lambda i: (i, 0)),
            pl.BlockSpec((bm, 128), lambda i: (i, 0)),
        ],
        out_shape=[
            jax.ShapeDtypeStruct((N, OUT), jnp.float32),
            jax.ShapeDtypeStruct((N, OUT), jnp.float32),
            jax.ShapeDtypeStruct((N, 128), jnp.float32),
        ],
    )(x, w1t, b1, w2t, b2, w3t, b3, pm, we)


def _gath_body(stab_hbm, dtab_hbm, a2w_hbm, srcp_hbm, dstp_hbm,
               gs_o, gdy_o, p1a_o, p2p_o, wrow_o,
               idx_s, idx_d, widx, srb, drb, wvb,
               gsb, gdb, p1a, p2p, sem):
    c = lax.axis_index("c")
    s = lax.axis_index("s")
    wid = s * 2 + c
    base = wid * EPW

    def chunk(ch, carry):
        e0 = base + ch * CH
        pltpu.sync_copy(srcp_hbm.at[pl.ds(e0, CH)], idx_s)
        pltpu.sync_copy(dstp_hbm.at[pl.ds(e0, CH)], idx_d)
        for g in range(CH // 16):
            sl = pl.ds(g * 16, 16)
            widx[sl] = lax.shift_right_logical(idx_s[sl] * NP + idx_d[sl], 9)
        cps = [pltpu.async_copy(stab_hbm.at[idx_s], srb, sem),
               pltpu.async_copy(dtab_hbm.at[idx_d], drb, sem),
               pltpu.async_copy(a2w_hbm.at[widx], wvb, sem)]
        for cp in cps:
            cp.wait()
        for e in range(CH):
            sid = srb[e, pl.ds(64, 16)]
            did = drb[e, pl.ds(48, 16)]
            a1v = jnp.where(sid == did, srb[e, pl.ds(48, 16)], 1.0)
            gsb[e] = srb[e, pl.ds(0, 16)]
            gdb[e] = drb[e, pl.ds(0, 16)]
            p1a[e] = srb[e, pl.ds(16, 16)] * drb[e, pl.ds(16, 16)] * a1v
            p2p[e] = srb[e, pl.ds(32, 16)] * drb[e, pl.ds(32, 16)]
        pltpu.sync_copy(gsb, gs_o.at[pl.ds(e0, CH)])
        pltpu.sync_copy(gdb, gdy_o.at[pl.ds(e0, CH)])
        pltpu.sync_copy(p1a, p1a_o.at[pl.ds(e0, CH)])
        pltpu.sync_copy(p2p, p2p_o.at[pl.ds(e0, CH)])
        pltpu.sync_copy(wvb, wrow_o.at[pl.ds(e0, CH)])
        return carry

    lax.fori_loop(0, NCH, chunk, 0)


def _gath(stab, dtab, a2w, srcp, dstp):
    f = pl.kernel(
        _gath_body,
        mesh=_sc_mesh(),
        out_type=[
            jax.ShapeDtypeStruct((EP, 16), jnp.float32),
            jax.ShapeDtypeStruct((EP, 16), jnp.float32),
            jax.ShapeDtypeStruct((EP, 16), jnp.float32),
            jax.ShapeDtypeStruct((EP, 16), jnp.float32),
            jax.ShapeDtypeStruct((EP, 128), jnp.int32),
        ],
        scratch_types=[
            pltpu.VMEM((CH,), jnp.int32),
            pltpu.VMEM((CH,), jnp.int32),
            pltpu.VMEM((CH,), jnp.int32),
            pltpu.VMEM((CH, 128), jnp.float32),
            pltpu.VMEM((CH, 128), jnp.float32),
            pltpu.VMEM((CH, 128), jnp.int32),
            pltpu.VMEM((CH, 16), jnp.float32),
            pltpu.VMEM((CH, 16), jnp.float32),
            pltpu.VMEM((CH, 16), jnp.float32),
            pltpu.VMEM((CH, 16), jnp.float32),
            pltpu.SemaphoreType.DMA,
        ],
    )
    return f(stab, dtab, a2w, srcp, dstp)


def _pack_body(lg, y, p1b, p2b, dgb, s_o, d_o, *, bm):
    i = pl.program_id(0)
    idv = (jax.lax.broadcasted_iota(jnp.int32, (bm, 16), 0) + i * bm).astype(
        jnp.float32)
    z = jnp.zeros((bm, 16), jnp.float32)
    s_o[...] = jnp.concatenate(
        [lg[...], p1b[...], p2b[...], dgb[...], idv, z, z, z], axis=1)
    d_o[...] = jnp.concatenate(
        [y[...], p1b[...], p2b[...], idv, z, z, z, z], axis=1)


def _pack(logits, y, p116, p216, dg16):
    bm = 1000
    return pl.pallas_call(
        functools.partial(_pack_body, bm=bm),
        grid=(N // bm,),
        in_specs=[pl.BlockSpec((bm, 16), lambda i: (i, 0)) for _ in range(5)],
        out_specs=[pl.BlockSpec((bm, 128), lambda i: (i, 0)),
                   pl.BlockSpec((bm, 128), lambda i: (i, 0))],
        out_shape=[jax.ShapeDtypeStruct((N, 128), jnp.float32),
                   jax.ShapeDtypeStruct((N, 128), jnp.float32)],
    )(logits, y, p116, p216, dg16)


def _ewstats_body(gs, gdy, s_o, q_o, *, bm):
    i = pl.program_id(0)
    ew = jnp.sum(gs[...] * gdy[...], axis=1, keepdims=True)
    eidx = jax.lax.broadcasted_iota(jnp.int32, (bm, 1), 0) + i * bm
    ewm = jnp.where(eidx < E, ew, 0.0)
    s_o[pl.ds(i, 1), :] = jnp.broadcast_to(
        jnp.sum(ewm, axis=0, keepdims=True), (1, 128))
    q_o[pl.ds(i, 1), :] = jnp.broadcast_to(
        jnp.sum(ewm * ewm, axis=0, keepdims=True), (1, 128))


def _ewstats(gs, gdy, bm=8192):
    nb = EP // bm
    return pl.pallas_call(
        functools.partial(_ewstats_body, bm=bm),
        grid=(nb,),
        in_specs=[pl.BlockSpec((bm, 16), lambda i: (i, 0)),
                  pl.BlockSpec((bm, 16), lambda i: (i, 0))],
        out_specs=[pl.BlockSpec((nb, 128), lambda i: (0, 0)),
                   pl.BlockSpec((nb, 128), lambda i: (0, 0))],
        out_shape=[jax.ShapeDtypeStruct((nb, 128), jnp.float32),
                   jax.ShapeDtypeStruct((nb, 128), jnp.float32)],
    )(gs, gdy)


def _mfin_body(gs, gdy, p1a, p2p, ssp, dsp, wrow, ssv, m1_o, m2_o, *, bm):
    i = pl.program_id(0)
    ew = jnp.sum(gs[...] * gdy[...], axis=1, keepdims=True)
    ewn = ew * ssv[0:1, 0:1] + ssv[0:1, 1:2]
    flat = ssp[:, 0:1] * NP + dsp[:, 0:1]
    lanes = jax.lax.broadcasted_iota(jnp.int32, (bm, 128), 1)
    lanesel = lax.shift_right_logical(flat, 2) & 127
    wsel = jnp.where(lanes == lanesel, wrow[...], 0)
    word = jnp.sum(wsel, axis=1, keepdims=True)
    byteoff = (flat & 3) * 8
    byte = lax.shift_right_logical(word, byteoff) & 255
    a2f = (byte > 0).astype(jnp.float32)
    eidx = jax.lax.broadcasted_iota(jnp.int32, (bm, 1), 0) + i * bm
    emask = jnp.where(eidx < E, 1.0, 0.0)
    m1_o[...] = p1a[...] * (ewn * emask)
    m2_o[...] = p2p[...] * (ewn * emask * a2f)


def _mfin(gs, gdy, p1a, p2p, ssp, dsp, wrow, ss, bm=4096):
    nb = EP // bm
    return pl.pallas_call(
        functools.partial(_mfin_body, bm=bm),
        grid=(nb,),
        in_specs=[pl.BlockSpec((bm, 16), lambda i: (i, 0)),
                  pl.BlockSpec((bm, 16), lambda i: (i, 0)),
                  pl.BlockSpec((bm, 16), lambda i: (i, 0)),
                  pl.BlockSpec((bm, 16), lambda i: (i, 0)),
                  pl.BlockSpec((bm, 1), lambda i: (i, 0)),
                  pl.BlockSpec((bm, 1), lambda i: (i, 0)),
                  pl.BlockSpec((bm, 128), lambda i: (i, 0)),
                  pl.BlockSpec((1, 128), lambda i: (0, 0))],
        out_specs=[pl.BlockSpec((bm, 16), lambda i: (i, 0)),
                   pl.BlockSpec((bm, 16), lambda i: (i, 0))],
        out_shape=[jax.ShapeDtypeStruct((EP, 16), jnp.float32),
                   jax.ShapeDtypeStruct((EP, 16), jnp.float32)],
    )(gs, gdy, p1a, p2p, ssp, dsp, wrow, ss)


PCH = 64


def _prop_body(tab_hbm, srcp_hbm, dstp_hbm, m1sp_hbm, m2sp_hbm, zhbm,
               out1, out2, idx_d, idx_s, mrows, rowb, scaled, acc, sem):
    c = lax.axis_index("c")
    s = lax.axis_index("s")
    t0 = s * 640
    for z in range(8):
        @pl.when(t0 + z * 80 < N)
        def _():
            pltpu.sync_copy(zhbm, acc.at[pl.ds(t0 + z * 80, 80)])
    plsc.subcore_barrier()

    def chunk(ch, carry):
        e0 = s * ET + ch * PCH
        pltpu.sync_copy(srcp_hbm.at[pl.ds(e0, PCH)], idx_s)
        pltpu.sync_copy(dstp_hbm.at[pl.ds(e0, PCH)], idx_d)

        @pl.when(c == 0)
        def _():
            pltpu.sync_copy(m1sp_hbm.at[pl.ds(e0, PCH)], mrows)

        @pl.when(c == 1)
        def _():
            pltpu.sync_copy(m2sp_hbm.at[pl.ds(e0, PCH)], mrows)

        pltpu.async_copy(tab_hbm.at[idx_d], rowb, sem).wait()
        for e in range(PCH):
            me = mrows[e]
            for q in range(8):
                scaled[e, pl.ds(q * 16, 16)] = rowb[e, pl.ds(q * 16, 16)] * me
        pltpu.sync_copy(scaled, acc.at[idx_s], add=True)
        return carry

    lax.fori_loop(0, ET // PCH, chunk, 0)
    plsc.subcore_barrier()
    for z in range(8):
        r0_ = t0 + z * 80

        @pl.when((c == 0) & (r0_ < N))
        def _():
            pltpu.sync_copy(acc.at[pl.ds(r0_, 80)], out1.at[pl.ds(r0_, 80)])

        @pl.when((c == 1) & (r0_ < N))
        def _():
            pltpu.sync_copy(acc.at[pl.ds(r0_, 80)], out2.at[pl.ds(r0_, 80)])


def _prop_kernel(tab, srcp, dstp, m1sp, m2sp):
    f = pl.kernel(
        _prop_body,
        mesh=_sc_mesh(),
        out_type=[
            jax.ShapeDtypeStruct((N, 128), jnp.float32),
            jax.ShapeDtypeStruct((N, 128), jnp.float32),
        ],
        scratch_types=[
            pltpu.VMEM((PCH,), jnp.int32),
            pltpu.VMEM((PCH,), jnp.int32),
            pltpu.VMEM((PCH, 16), jnp.float32),
            pltpu.VMEM((PCH, 128), jnp.float32),
            pltpu.VMEM((PCH, 128), jnp.float32),
            pltpu.VMEM_SHARED((N, 128), jnp.float32),
            pltpu.SemaphoreType.DMA,
        ],
    )
    zhbm = jnp.zeros((80, 128), jnp.float32)
    return f(tab, srcp, dstp, m1sp, m2sp, zhbm)


def _relu2_body(a, b, pk_o, ao, bo):
    q1 = jnp.maximum(a[..., :HID], 0.0)
    q2 = jnp.maximum(b[..., :HID], 0.0)
    pk_o[...] = jnp.concatenate([q1, q2], axis=1)
    ao[...] = q1
    bo[...] = q2


def _relu2(a, b):
    bm = 1000
    return pl.pallas_call(
        _relu2_body,
        grid=(N // bm,),
        in_specs=[pl.BlockSpec((bm, 128), lambda i: (i, 0)),
                  pl.BlockSpec((bm, 128), lambda i: (i, 0))],
        out_specs=[pl.BlockSpec((bm, 128), lambda i: (i, 0)),
                   pl.BlockSpec((bm, HID), lambda i: (i, 0)),
                   pl.BlockSpec((bm, HID), lambda i: (i, 0))],
        out_shape=[jax.ShapeDtypeStruct((N, 128), jnp.float32),
                   jax.ShapeDtypeStruct((N, HID), jnp.float32),
                   jax.ShapeDtypeStruct((N, HID), jnp.float32)],
    )(a, b)


def _cls_body(r0, q1, q2, q3, q4, w0, w1, w2, w3, w4, out):
    z = jnp.dot(r0[..., :HID], w0[...], preferred_element_type=jnp.float32)
    z += jnp.dot(q1[...], w1[...], preferred_element_type=jnp.float32)
    z += jnp.dot(q2[...], w2[...], preferred_element_type=jnp.float32)
    z += jnp.dot(jnp.maximum(q3[...], 0.0), w3[...], preferred_element_type=jnp.float32)
    z += jnp.dot(jnp.maximum(q4[...], 0.0), w4[...], preferred_element_type=jnp.float32)
    z = z - jnp.max(z, axis=1, keepdims=True)
    ez = jnp.exp(z)
    p = ez / jnp.sum(ez, axis=1, keepdims=True)
    p2_ = p - jnp.max(p, axis=1, keepdims=True)
    out[...] = p2_ - jnp.log(jnp.sum(jnp.exp(p2_), axis=1, keepdims=True))


def _classify(r0, q1, q2, q3, q4, w0, w1, w2, w3, w4):
    bm = 1000
    g = N // bm
    return pl.pallas_call(
        _cls_body,
        grid=(g,),
        in_specs=[
            pl.BlockSpec((bm, 128), lambda i: (i, 0)),
            pl.BlockSpec((bm, HID), lambda i: (i, 0)),
            pl.BlockSpec((bm, HID), lambda i: (i, 0)),
            pl.BlockSpec((bm, 128), lambda i: (i, 0)),
            pl.BlockSpec((bm, 128), lambda i: (i, 0)),
            pl.BlockSpec((HID, OUT), lambda i: (0, 0)),
            pl.BlockSpec((HID, OUT), lambda i: (0, 0)),
            pl.BlockSpec((HID, OUT), lambda i: (0, 0)),
            pl.BlockSpec((128, OUT), lambda i: (0, 0)),
            pl.BlockSpec((128, OUT), lambda i: (0, 0)),
        ],
        out_specs=pl.BlockSpec((bm, OUT), lambda i: (i, 0)),
        out_shape=jax.ShapeDtypeStruct((N, OUT), jnp.float32),
    )(r0, q1, q2, q3, q4, w0, w1, w2, w3, w4)


def kernel(x, edge_index, w_embed, w_classify, parsing, mw1, mb1, mw2, mb2, mw3, mb3):
    src = edge_index[0]
    dst = edge_index[1]
    srcp = jnp.pad(src, (0, EP - E))
    dstp = jnp.pad(dst, (0, EP - E))

    A = _cvt_bf16(_abuild(srcp, dstp).reshape(NP, NP))
    a2_i8, p1c, p2c, dgc = _adj_structure(A, np_=NP)
    p1t16 = p1c[:N, :16]
    p2t16 = p2c[:N, :16]
    dg16 = dgc[:N, :16]
    a2w2 = lax.bitcast_convert_type(a2_i8.reshape(NP * NP // 512, 128, 4), jnp.int32)

    pm = jnp.maximum(2.0 * parsing, 0.0)
    logits, Y, r0 = _mlp(x, mw1.T, mb1.reshape(1, 512), mw2.T, mb2.reshape(1, HID),
                         mw3.T, mb3.reshape(1, OUT), pm, w_embed)

    stab, dtab = _pack(logits, Y, p1t16, p2t16, dg16)
    gs, gdy, p1a, p2p, wrow = _gath(stab, dtab, a2w2, srcp, dstp)

    s_o, q_o = _ewstats(gs, gdy)
    S = jnp.sum(s_o[:, 0])
    Q = jnp.sum(q_o[:, 0])
    mean = S / E
    var = (Q - S * S / E) / (E - 1)
    scale = jnp.sqrt(1e-4 / var)
    shift = 1.0 - mean * scale
    ss = jnp.zeros((1, 128), jnp.float32).at[0, 0].set(scale).at[0, 1].set(shift)

    m1sp, m2sp = _mfin(gs, gdy, p1a, p2p,
                       srcp.reshape(EP, 1), dstp.reshape(EP, 1), wrow, ss)

    r1_raw, r2_raw = _prop_kernel(r0, srcp, dstp, m1sp, m2sp)
    qpack, q1, q2 = _relu2(r1_raw, r2_raw)
    q3, q4 = _prop_kernel(qpack, srcp, dstp, m1sp, m2sp)

    return _classify(r0, q1, q2, q3, q4,
                     w_classify[0:64], w_classify[64:128], w_classify[128:192],
                     w_classify[192:320], w_classify[320:448])


# speedup vs baseline: 1.1919x; 1.1919x over previous
import functools

import jax
import jax.numpy as jnp
from jax import lax
from jax.experimental import pallas as pl
from jax.experimental.pallas import tpu as pltpu
from jax.experimental.pallas import tpu_sc as plsc

N = 10000
E = 160000
D_IN = 128
HID = 64
OUT = 16
NP = 10240
EP = 163840
NW = 32
EPW = EP // NW
CH = 128
NCH = EPW // CH
ET = EP // 16
BR = 160
NBLK = NP // BR


def _sc_mesh():
    return plsc.VectorSubcoreMesh(core_axis_name="c", subcore_axis_name="s")


def _abuild_body(srcp_hbm, dstp_hbm, zhbm, ones_hbm, a_hbm, srcv, dstv, idxb, valb, spa):
    c = lax.axis_index("c")
    s = lax.axis_index("s")
    lane = lax.iota(jnp.int32, 16)
    ebase = s * ET
    pltpu.sync_copy(srcp_hbm.at[pl.ds(ebase, ET)], srcv)
    pltpu.sync_copy(dstp_hbm.at[pl.ds(ebase, ET)], dstv)
    pltpu.sync_copy(ones_hbm, valb)
    trash = jnp.int32(BR * NP)

    def block_body(ib, carry):
        blk = ib * 2 + c
        lo = blk * BR
        r0 = s * 10
        pltpu.sync_copy(zhbm, spa.at[pl.ds(r0 * NP, 10 * NP)])

        @pl.when(s == 0)
        def _():
            pltpu.sync_copy(zhbm.at[pl.ds(0, NP)], spa.at[pl.ds(BR * NP, NP)])

        plsc.subcore_barrier()

        def chunk_scan(q, carry2):
            for g in range(CH // 16):
                sl = pl.ds(q * CH + g * 16, 16)
                sv = srcv[sl]
                dv = dstv[sl]
                rel = sv - lo
                inb = ((rel >= 0) & (rel < BR)
                       & ((ebase + q * CH + g * 16 + lane) < E))
                idxb[pl.ds(g * 16, 16)] = jnp.where(inb, rel * NP + dv, trash)
            pltpu.sync_copy(valb, spa.at[idxb], add=True)
            return carry2

        lax.fori_loop(0, ET // CH, chunk_scan, 0)
        plsc.subcore_barrier()
        pltpu.sync_copy(spa.at[pl.ds(r0 * NP, 10 * NP)],
                        a_hbm.at[pl.ds((lo + r0) * NP, 10 * NP)])
        plsc.subcore_barrier()
        return carry

    lax.fori_loop(0, NBLK // 2, block_body, 0)


def _abuild(srcp, dstp):
    f = pl.kernel(
        _abuild_body,
        mesh=_sc_mesh(),
        out_type=[jax.ShapeDtypeStruct((NP * NP,), jnp.float32)],
        scratch_types=[
            pltpu.VMEM((ET,), jnp.int32),
            pltpu.VMEM((ET,), jnp.int32),
            pltpu.VMEM((CH,), jnp.int32),
            pltpu.VMEM((CH,), jnp.float32),
            pltpu.VMEM_SHARED(((BR + 1) * NP,), jnp.float32),
        ],
    )
    zhbm = jnp.zeros((10 * NP,), jnp.float32)
    ones_hbm = jnp.ones((CH,), jnp.float32)
    return f(srcp, dstp, zhbm, ones_hbm)[0]


def _cvt_body(a32, ab):
    ab[...] = a32[...].astype(jnp.bfloat16)


def _cvt_bf16(a32):
    bm, bn = 512, 2048
    return pl.pallas_call(
        _cvt_body,
        grid=(NP // bm, NP // bn),
        in_specs=[pl.BlockSpec((bm, bn), lambda i, j: (i, j))],
        out_specs=pl.BlockSpec((bm, bn), lambda i, j: (i, j)),
        out_shape=jax.ShapeDtypeStruct((NP, NP), jnp.bfloat16),
    )(a32)


def _adj_body(aij, al, ar, a2o, p1o, p2o, dgo, acc, *, bm, bn):
    i, j, k = pl.program_id(0), pl.program_id(1), pl.program_id(2)
    nj, nk = pl.num_programs(1), pl.num_programs(2)

    @pl.when(k == 0)
    def _():
        acc[...] = jnp.zeros_like(acc)

    acc[...] += jnp.dot(al[...], ar[...], preferred_element_type=jnp.float32)

    @pl.when((j == 0) & (k == 0))
    def _():
        p1o[...] = jnp.zeros_like(p1o)
        p2o[...] = jnp.zeros_like(p2o)
        dgo[...] = jnp.zeros_like(dgo)

    @pl.when(k == nk - 1)
    def _():
        a = aij[...].astype(jnp.float32)
        rows = jax.lax.broadcasted_iota(jnp.int32, (bm, bn), 0) + i * bm
        cols = jax.lax.broadcasted_iota(jnp.int32, (bm, bn), 1) + j * bn
        dmask = (rows == cols).astype(jnp.float32)
        a1 = ((a - dmask) > 0.5).astype(jnp.float32)
        a2 = (acc[...] - a - dmask) > 0.5
        a2o[...] = a2.astype(jnp.int8)
        p1o[...] += jnp.broadcast_to(jnp.sum(a1, axis=1, keepdims=True), p1o.shape)
        p2o[...] += jnp.broadcast_to(
            jnp.sum(a2.astype(jnp.float32), axis=1, keepdims=True), p2o.shape)

        @pl.when(i == j)
        def _():
            dgo[...] += jnp.broadcast_to(
                jnp.sum(a * dmask, axis=1, keepdims=True), dgo.shape)

        @pl.when(j == nj - 1)
        def _():
            d1 = p1o[...]
            p1o[...] = jnp.where(d1 > 0.5, jax.lax.rsqrt(jnp.maximum(d1, 1e-30)), 0.0)
            d2 = p2o[...]
            p2o[...] = jnp.where(d2 > 0.5, jax.lax.rsqrt(jnp.maximum(d2, 1e-30)), 0.0)
            dgo[...] = (dgo[...] - 1.0 > 0.5).astype(jnp.float32)


def _adj_structure(a_bf16, *, np_, bm=1024, bn=1024, bk=512, interpret=False):
    nbi, nbj, nbk = np_ // bm, np_ // bn, np_ // bk
    return pl.pallas_call(
        functools.partial(_adj_body, bm=bm, bn=bn),
        grid=(nbi, nbj, nbk),
        in_specs=[
            pl.BlockSpec((bm, bn), lambda i, j, k: (i, j)),
            pl.BlockSpec((bm, bk), lambda i, j, k: (i, k)),
            pl.BlockSpec((bk, bn), lambda i, j, k: (k, j)),
        ],
        out_specs=[
            pl.BlockSpec((bm, bn), lambda i, j, k: (i, j)),
            pl.BlockSpec((bm, 128), lambda i, j, k: (i, 0)),
            pl.BlockSpec((bm, 128), lambda i, j, k: (i, 0)),
            pl.BlockSpec((bm, 128), lambda i, j, k: (i, 0)),
        ],
        out_shape=[
            jax.ShapeDtypeStruct((np_, np_), jnp.int8),
            jax.ShapeDtypeStruct((np_, 128), jnp.float32),
            jax.ShapeDtypeStruct((np_, 128), jnp.float32),
            jax.ShapeDtypeStruct((np_, 128), jnp.float32),
        ],
        scratch_shapes=[pltpu.VMEM((bm, bn), jnp.float32)],
        compiler_params=pltpu.CompilerParams(
            dimension_semantics=("parallel", "parallel", "arbitrary")),
        interpret=interpret,
    )(a_bf16, a_bf16, a_bf16)


def _mlp_body(xb, w1t, b1, w2t, b2, w3t, b3, pm, we, lg_o, y_o, r0_o):
    xv = xb[...]
    h1 = jnp.maximum(
        jnp.dot(xv, w1t[...], preferred_element_type=jnp.float32) + b1[...], 0.0)
    h2 = jnp.maximum(
        jnp.dot(h1, w2t[...], preferred_element_type=jnp.float32) + b2[...], 0.0)
    lg = jnp.dot(h2, w3t[...], preferred_element_type=jnp.float32) + b3[...]
    lg_o[...] = lg
    y_o[...] = jnp.dot(lg, pm[...], preferred_element_type=jnp.float32)
    r0 = jnp.maximum(
        jnp.dot(xv, we[...], preferred_element_type=jnp.float32), 0.0)
    r0_o[...] = jnp.concatenate(
        [r0, jnp.zeros((r0.shape[0], 128 - HID), jnp.float32)], axis=1)


def _mlp(x, w1t, b1, w2t, b2, w3t, b3, pm, we):
    bm = 1000
    g = N // bm
    return pl.pallas_call(
        _mlp_body,
        grid=(g,),
        in_specs=[
            pl.BlockSpec((bm, D_IN), lambda i: (i, 0)),
            pl.BlockSpec((D_IN, 512), lambda i: (0, 0)),
            pl.BlockSpec((1, 512), lambda i: (0, 0)),
            pl.BlockSpec((512, HID), lambda i: (0, 0)),
            pl.BlockSpec((1, HID), lambda i: (0, 0)),
            pl.BlockSpec((HID, OUT), lambda i: (0, 0)),
            pl.BlockSpec((1, OUT), lambda i: (0, 0)),
            pl.BlockSpec((OUT, OUT), lambda i: (0, 0)),
            pl.BlockSpec((D_IN, HID), lambda i: (0, 0)),
        ],
        out_specs=[
            pl.BlockSpec((bm, OUT), lambda i: (i, 0)),
            pl.BlockSpec((bm, OUT), lambda i: (i, 0)),
            pl.BlockSpec((bm, 128), lambda i: (i, 0)),
        ],
        out_shape=[
            jax.ShapeDtypeStruct((N, OUT), jnp.float32),
            jax.ShapeDtypeStruct((N, OUT), jnp.float32),
            jax.ShapeDtypeStruct((N, 128), jnp.float32),
        ],
    )(x, w1t, b1, w2t, b2, w3t, b3, pm, we)


def _gath_body(stab_hbm, dtab_hbm, a2w_hbm, srcp_hbm, dstp_hbm,
               gs_o, gdy_o, p1a_o, p2p_o, wrow_o,
               idx_s, idx_d, widx, srb, drb, wvb,
               gsb, gdb, p1a, p2p, sem):
    c = lax.axis_index("c")
    s = lax.axis_index("s")
    wid = s * 2 + c
    base = wid * EPW

    def chunk(ch, carry):
        e0 = base + ch * CH
        pltpu.sync_copy(srcp_hbm.at[pl.ds(e0, CH)], idx_s)
        pltpu.sync_copy(dstp_hbm.at[pl.ds(e0, CH)], idx_d)
        for g in range(CH // 16):
            sl = pl.ds(g * 16, 16)
            widx[sl] = lax.shift_right_logical(idx_s[sl] * NP + idx_d[sl], 9)
        cps = [pltpu.async_copy(stab_hbm.at[idx_s], srb, sem),
               pltpu.async_copy(dtab_hbm.at[idx_d], drb, sem),
               pltpu.async_copy(a2w_hbm.at[widx], wvb, sem)]
        for cp in cps:
            cp.wait()
        for e in range(CH):
            sid = srb[e, pl.ds(64, 16)]
            did = drb[e, pl.ds(48, 16)]
            a1v = jnp.where(sid == did, srb[e, pl.ds(48, 16)], 1.0)
            gsb[e] = srb[e, pl.ds(0, 16)]
            gdb[e] = drb[e, pl.ds(0, 16)]
            p1a[e] = srb[e, pl.ds(16, 16)] * drb[e, pl.ds(16, 16)] * a1v
            p2p[e] = srb[e, pl.ds(32, 16)] * drb[e, pl.ds(32, 16)]
        pltpu.sync_copy(gsb, gs_o.at[pl.ds(e0, CH)])
        pltpu.sync_copy(gdb, gdy_o.at[pl.ds(e0, CH)])
        pltpu.sync_copy(p1a, p1a_o.at[pl.ds(e0, CH)])
        pltpu.sync_copy(p2p, p2p_o.at[pl.ds(e0, CH)])
        pltpu.sync_copy(wvb, wrow_o.at[pl.ds(e0, CH)])
        return carry

    lax.fori_loop(0, NCH, chunk, 0)


def _gath(stab, dtab, a2w, srcp, dstp):
    f = pl.kernel(
        _gath_body,
        mesh=_sc_mesh(),
        out_type=[
            jax.ShapeDtypeStruct((EP, 16), jnp.float32),
            jax.ShapeDtypeStruct((EP, 16), jnp.float32),
            jax.ShapeDtypeStruct((EP, 16), jnp.float32),
            jax.ShapeDtypeStruct((EP, 16), jnp.float32),
            jax.ShapeDtypeStruct((EP, 128), jnp.int32),
        ],
        scratch_types=[
            pltpu.VMEM((CH,), jnp.int32),
            pltpu.VMEM((CH,), jnp.int32),
            pltpu.VMEM((CH,), jnp.int32),
            pltpu.VMEM((CH, 128), jnp.float32),
            pltpu.VMEM((CH, 128), jnp.float32),
            pltpu.VMEM((CH, 128), jnp.int32),
            pltpu.VMEM((CH, 16), jnp.float32),
            pltpu.VMEM((CH, 16), jnp.float32),
            pltpu.VMEM((CH, 16), jnp.float32),
            pltpu.VMEM((CH, 16), jnp.float32),
            pltpu.SemaphoreType.DMA,
        ],
    )
    return f(stab, dtab, a2w, srcp, dstp)


def _pack_body(lg, y, p1b, p2b, dgb, s_o, d_o, *, bm):
    i = pl.program_id(0)
    idv = (jax.lax.broadcasted_iota(jnp.int32, (bm, 16), 0) + i * bm).astype(
        jnp.float32)
    z = jnp.zeros((bm, 16), jnp.float32)
    s_o[...] = jnp.concatenate(
        [lg[...], p1b[...], p2b[...], dgb[...], idv, z, z, z], axis=1)
    d_o[...] = jnp.concatenate(
        [y[...], p1b[...], p2b[...], idv, z, z, z, z], axis=1)


def _pack(logits, y, p116, p216, dg16):
    bm = 1000
    return pl.pallas_call(
        functools.partial(_pack_body, bm=bm),
        grid=(N // bm,),
        in_specs=[pl.BlockSpec((bm, 16), lambda i: (i, 0)) for _ in range(5)],
        out_specs=[pl.BlockSpec((bm, 128), lambda i: (i, 0)),
                   pl.BlockSpec((bm, 128), lambda i: (i, 0))],
        out_shape=[jax.ShapeDtypeStruct((N, 128), jnp.float32),
                   jax.ShapeDtypeStruct((N, 128), jnp.float32)],
    )(logits, y, p116, p216, dg16)


def _ewstats_body(gs, gdy, s_o, q_o, *, bm):
    i = pl.program_id(0)
    ew = jnp.sum(gs[...] * gdy[...], axis=1, keepdims=True)
    eidx = jax.lax.broadcasted_iota(jnp.int32, (bm, 1), 0) + i * bm
    ewm = jnp.where(eidx < E, ew, 0.0)
    s_o[pl.ds(i, 1), :] = jnp.broadcast_to(
        jnp.sum(ewm, axis=0, keepdims=True), (1, 128))
    q_o[pl.ds(i, 1), :] = jnp.broadcast_to(
        jnp.sum(ewm * ewm, axis=0, keepdims=True), (1, 128))


def _ewstats(gs, gdy, bm=8192):
    nb = EP // bm
    return pl.pallas_call(
        functools.partial(_ewstats_body, bm=bm),
        grid=(nb,),
        in_specs=[pl.BlockSpec((bm, 16), lambda i: (i, 0)),
                  pl.BlockSpec((bm, 16), lambda i: (i, 0))],
        out_specs=[pl.BlockSpec((nb, 128), lambda i: (0, 0)),
                   pl.BlockSpec((nb, 128), lambda i: (0, 0))],
        out_shape=[jax.ShapeDtypeStruct((nb, 128), jnp.float32),
                   jax.ShapeDtypeStruct((nb, 128), jnp.float32)],
    )(gs, gdy)


def _mfin_body(gs, gdy, p1a, p2p, ssp, dsp, wrow, ssv, m1_o, m2_o, *, bm):
    i = pl.program_id(0)
    ew = jnp.sum(gs[...] * gdy[...], axis=1, keepdims=True)
    ewn = ew * ssv[0:1, 0:1] + ssv[0:1, 1:2]
    flat = ssp[:, 0:1] * NP + dsp[:, 0:1]
    lanes = jax.lax.broadcasted_iota(jnp.int32, (bm, 128), 1)
    lanesel = lax.shift_right_logical(flat, 2) & 127
    wsel = jnp.where(lanes == lanesel, wrow[...], 0)
    word = jnp.sum(wsel, axis=1, keepdims=True)
    byteoff = (flat & 3) * 8
    byte = lax.shift_right_logical(word, byteoff) & 255
    a2f = (byte > 0).astype(jnp.float32)
    eidx = jax.lax.broadcasted_iota(jnp.int32, (bm, 1), 0) + i * bm
    emask = jnp.where(eidx < E, 1.0, 0.0)
    m1_o[...] = p1a[...] * (ewn * emask)
    m2_o[...] = p2p[...] * (ewn * emask * a2f)


def _mfin(gs, gdy, p1a, p2p, ssp, dsp, wrow, ss, bm=4096):
    nb = EP // bm
    return pl.pallas_call(
        functools.partial(_mfin_body, bm=bm),
        grid=(nb,),
        in_specs=[pl.BlockSpec((bm, 16), lambda i: (i, 0)),
                  pl.BlockSpec((bm, 16), lambda i: (i, 0)),
                  pl.BlockSpec((bm, 16), lambda i: (i, 0)),
                  pl.BlockSpec((bm, 16), lambda i: (i, 0)),
                  pl.BlockSpec((bm, 1), lambda i: (i, 0)),
                  pl.BlockSpec((bm, 1), lambda i: (i, 0)),
                  pl.BlockSpec((bm, 128), lambda i: (i, 0)),
                  pl.BlockSpec((1, 128), lambda i: (0, 0))],
        out_specs=[pl.BlockSpec((bm, 16), lambda i: (i, 0)),
                   pl.BlockSpec((bm, 16), lambda i: (i, 0))],
        out_shape=[jax.ShapeDtypeStruct((EP, 16), jnp.float32),
                   jax.ShapeDtypeStruct((EP, 16), jnp.float32)],
    )(gs, gdy, p1a, p2p, ssp, dsp, wrow, ss)


PCH = 64


def _prop_body(tab_hbm, srcp_hbm, dstp_hbm, m1sp_hbm, m2sp_hbm, zhbm,
               out1, out2, idx_d, idx_s, mrows, rowb, scaled, acc, sem):
    c = lax.axis_index("c")
    s = lax.axis_index("s")
    t0 = s * 640
    for z in range(8):
        @pl.when(t0 + z * 80 < N)
        def _():
            pltpu.sync_copy(zhbm, acc.at[pl.ds(t0 + z * 80, 80)])
    plsc.subcore_barrier()

    def chunk(ch, carry):
        e0 = s * ET + ch * PCH
        pltpu.sync_copy(srcp_hbm.at[pl.ds(e0, PCH)], idx_s)
        pltpu.sync_copy(dstp_hbm.at[pl.ds(e0, PCH)], idx_d)

        @pl.when(c == 0)
        def _():
            pltpu.sync_copy(m1sp_hbm.at[pl.ds(e0, PCH)], mrows)

        @pl.when(c == 1)
        def _():
            pltpu.sync_copy(m2sp_hbm.at[pl.ds(e0, PCH)], mrows)

        pltpu.async_copy(tab_hbm.at[idx_d], rowb, sem).wait()
        for e in range(PCH):
            me = mrows[e]
            for q in range(8):
                scaled[e, pl.ds(q * 16, 16)] = rowb[e, pl.ds(q * 16, 16)] * me
        pltpu.sync_copy(scaled, acc.at[idx_s], add=True)
        return carry

    lax.fori_loop(0, ET // PCH, chunk, 0)
    plsc.subcore_barrier()
    for z in range(8):
        r0_ = t0 + z * 80

        @pl.when((c == 0) & (r0_ < N))
        def _():
            pltpu.sync_copy(acc.at[pl.ds(r0_, 80)], out1.at[pl.ds(r0_, 80)])

        @pl.when((c == 1) & (r0_ < N))
        def _():
            pltpu.sync_copy(acc.at[pl.ds(r0_, 80)], out2.at[pl.ds(r0_, 80)])


def _prop_kernel(tab, srcp, dstp, m1sp, m2sp):
    f = pl.kernel(
        _prop_body,
        mesh=_sc_mesh(),
        out_type=[
            jax.ShapeDtypeStruct((N, 128), jnp.float32),
            jax.ShapeDtypeStruct((N, 128), jnp.float32),
        ],
        scratch_types=[
            pltpu.VMEM((PCH,), jnp.int32),
            pltpu.VMEM((PCH,), jnp.int32),
            pltpu.VMEM((PCH, 16), jnp.float32),
            pltpu.VMEM((PCH, 128), jnp.float32),
            pltpu.VMEM((PCH, 128), jnp.float32),
            pltpu.VMEM_SHARED((N, 128), jnp.float32),
            pltpu.SemaphoreType.DMA,
        ],
    )
    zhbm = jnp.zeros((80, 128), jnp.float32)
    return f(tab, srcp, dstp, m1sp, m2sp, zhbm)


def _relu2_body(a, b, pk_o, ao, bo):
    q1 = jnp.maximum(a[..., :HID], 0.0)
    q2 = jnp.maximum(b[..., :HID], 0.0)
    pk_o[...] = jnp.concatenate([q1, q2], axis=1)
    ao[...] = q1
    bo[...] = q2


def _relu2(a, b):
    bm = 1000
    return pl.pallas_call(
        _relu2_body,
        grid=(N // bm,),
        in_specs=[pl.BlockSpec((bm, 128), lambda i: (i, 0)),
                  pl.BlockSpec((bm, 128), lambda i: (i, 0))],
        out_specs=[pl.BlockSpec((bm, 128), lambda i: (i, 0)),
                   pl.BlockSpec((bm, HID), lambda i: (i, 0)),
                   pl.BlockSpec((bm, HID), lambda i: (i, 0))],
        out_shape=[jax.ShapeDtypeStruct((N, 128), jnp.float32),
                   jax.ShapeDtypeStruct((N, HID), jnp.float32),
                   jax.ShapeDtypeStruct((N, HID), jnp.float32)],
    )(a, b)


def _cls_body(r0, q1, q2, q3, q4, w0, w1, w2, w3, w4, out):
    z = jnp.dot(r0[..., :HID], w0[...], preferred_element_type=jnp.float32)
    z += jnp.dot(q1[...], w1[...], preferred_element_type=jnp.float32)
    z += jnp.dot(q2[...], w2[...], preferred_element_type=jnp.float32)
    z += jnp.dot(jnp.maximum(q3[...], 0.0), w3[...], preferred_element_type=jnp.float32)
    z += jnp.dot(jnp.maximum(q4[...], 0.0), w4[...], preferred_element_type=jnp.float32)
    z = z - jnp.max(z, axis=1, keepdims=True)
    ez = jnp.exp(z)
    p = ez / jnp.sum(ez, axis=1, keepdims=True)
    p2_ = p - jnp.max(p, axis=1, keepdims=True)
    out[...] = p2_ - jnp.log(jnp.sum(jnp.exp(p2_), axis=1, keepdims=True))


def _classify(r0, q1, q2, q3, q4, w0, w1, w2, w3, w4):
    bm = 1000
    g = N // bm
    return pl.pallas_call(
        _cls_body,
        grid=(g,),
        in_specs=[
            pl.BlockSpec((bm, 128), lambda i: (i, 0)),
            pl.BlockSpec((bm, HID), lambda i: (i, 0)),
            pl.BlockSpec((bm, HID), lambda i: (i, 0)),
            pl.BlockSpec((bm, 128), lambda i: (i, 0)),
            pl.BlockSpec((bm, 128), lambda i: (i, 0)),
            pl.BlockSpec((HID, OUT), lambda i: (0, 0)),
            pl.BlockSpec((HID, OUT), lambda i: (0, 0)),
            pl.BlockSpec((HID, OUT), lambda i: (0, 0)),
            pl.BlockSpec((128, OUT), lambda i: (0, 0)),
            pl.BlockSpec((128, OUT), lambda i: (0, 0)),
        ],
        out_specs=pl.BlockSpec((bm, OUT), lambda i: (i, 0)),
        out_shape=jax.ShapeDtypeStruct((N, OUT), jnp.float32),
    )(r0, q1, q2, q3, q4, w0, w1, w2, w3, w4)


def kernel(x, edge_index, w_embed, w_classify, parsing, mw1, mb1, mw2, mb2, mw3, mb3):
    src = edge_index[0]
    dst = edge_index[1]
    srcp = jnp.pad(src, (0, EP - E))
    dstp = jnp.pad(dst, (0, EP - E))

    a32 = jnp.zeros((NP, NP), jnp.float32).at[src, dst].add(1.0)
    A = _cvt_bf16(a32)
    a2_i8, p1c, p2c, dgc = _adj_structure(A, np_=NP)
    p1t16 = p1c[:N, :16]
    p2t16 = p2c[:N, :16]
    dg16 = dgc[:N, :16]
    a2w2 = lax.bitcast_convert_type(a2_i8.reshape(NP * NP // 512, 128, 4), jnp.int32)

    pm = jnp.maximum(2.0 * parsing, 0.0)
    logits, Y, r0 = _mlp(x, mw1.T, mb1.reshape(1, 512), mw2.T, mb2.reshape(1, HID),
                         mw3.T, mb3.reshape(1, OUT), pm, w_embed)

    stab, dtab = _pack(logits, Y, p1t16, p2t16, dg16)
    gs, gdy, p1a, p2p, wrow = _gath(stab, dtab, a2w2, srcp, dstp)

    s_o, q_o = _ewstats(gs, gdy)
    S = jnp.sum(s_o[:, 0])
    Q = jnp.sum(q_o[:, 0])
    mean = S / E
    var = (Q - S * S / E) / (E - 1)
    scale = jnp.sqrt(1e-4 / var)
    shift = 1.0 - mean * scale
    ss = jnp.zeros((1, 128), jnp.float32).at[0, 0].set(scale).at[0, 1].set(shift)

    m1sp, m2sp = _mfin(gs, gdy, p1a, p2p,
                       srcp.reshape(EP, 1), dstp.reshape(EP, 1), wrow, ss)

    r1_raw, r2_raw = _prop_kernel(r0, srcp, dstp, m1sp, m2sp)
    qpack, q1, q2 = _relu2(r1_raw, r2_raw)
    q3, q4 = _prop_kernel(qpack, srcp, dstp, m1sp, m2sp)

    return _classify(r0, q1, q2, q3, q4,
                     w_classify[0:64], w_classify[64:128], w_classify[128:192],
                     w_classify[192:320], w_classify[320:448])
